# baseline (speedup 1.0000x reference)
import jax
import jax.numpy as jnp
from jax import lax
from jax.experimental import pallas as pl
from jax.experimental.pallas import tpu as pltpu
from jax.experimental.pallas import tpu_sc as plsc

_TOTAL_TOKENS = 32768
_BATCH = 16
_D_MODEL = 4096
_LANES = 16
_NCHUNK = _D_MODEL // _LANES
_ROWS_PER_CORE = _BATCH // 2


def _pooler_body(hs_hbm, lens_hbm, out_hbm, lens_v, row_v, sem):
    c = lax.axis_index("c")
    s = lax.axis_index("s")

    @pl.when(s < _ROWS_PER_CORE)
    def _():
        r = c * _ROWS_PER_CORE + s
        pltpu.sync_copy(lens_hbm, lens_v)
        last_idx = plsc.cumsum(lens_v[...]) - 1
        lane = lax.iota(jnp.int32, _LANES)
        row_idx = jnp.sum(jnp.where(lane == r, last_idx, 0))

        pltpu.async_copy(hs_hbm.at[pl.ds(row_idx, 1)], row_v, sem).wait()

        def acc_body(i, acc):
            v = row_v[0, pl.ds(i * _LANES, _LANES)]
            return acc + v * v

        acc = lax.fori_loop(0, _NCHUNK, acc_body, jnp.zeros((_LANES,), jnp.float32))
        ss = jnp.full((_LANES,), jnp.sum(acc), jnp.float32)

        seed = jnp.int32(0x5F3759DF) - (plsc.bitcast(ss, jnp.int32) >> 1)
        y = plsc.bitcast(seed, jnp.float32)
        for _unused in range(3):
            y = y * (1.5 - 0.5 * ss * y * y)
        norm = ss * y
        scale_v = jnp.where(norm > 1e-12, y, jnp.float32(1e12))

        def scale_body(i, carry):
            sl = pl.ds(i * _LANES, _LANES)
            row_v[0, sl] = row_v[0, sl] * scale_v
            return carry

        lax.fori_loop(0, _NCHUNK, scale_body, 0)
        pltpu.sync_copy(row_v, out_hbm.at[pl.ds(r, 1)])


def kernel(hidden_states, extend_seq_lens):
    mesh = plsc.VectorSubcoreMesh(
        core_axis_name="c", subcore_axis_name="s", num_cores=2, num_subcores=16
    )
    f = pl.kernel(
        _pooler_body,
        out_type=jax.ShapeDtypeStruct((_BATCH, _D_MODEL), jnp.float32),
        mesh=mesh,
        scratch_types=[
            pltpu.VMEM((_LANES,), jnp.int32),
            pltpu.VMEM((1, _D_MODEL), jnp.float32),
            pltpu.SemaphoreType.DMA,
        ],
        compiler_params=pltpu.CompilerParams(needs_layout_passes=False),
    )
    return f(hidden_states, extend_seq_lens)


# speedup vs baseline: 1.0219x; 1.0219x over previous
import jax
import jax.numpy as jnp
from jax import lax
from jax.experimental import pallas as pl
from jax.experimental.pallas import tpu as pltpu
from jax.experimental.pallas import tpu_sc as plsc

_TOTAL_TOKENS = 32768
_BATCH = 16
_D_MODEL = 4096
_LANES = 16
_NCHUNK = _D_MODEL // _LANES
_ROWS_PER_CORE = _BATCH // 2


def _pooler_body(hs_hbm, lens_hbm, out_hbm, lens_v, row_v, sem):
    c = lax.axis_index("c")
    s = lax.axis_index("s")

    @pl.when(s < _ROWS_PER_CORE)
    def _():
        r = c * _ROWS_PER_CORE + s
        pltpu.sync_copy(lens_hbm, lens_v)
        last_idx = plsc.cumsum(lens_v[...]) - 1
        lane = lax.iota(jnp.int32, _LANES)
        row_idx = jnp.sum(jnp.where(lane == r, last_idx, 0))

        pltpu.async_copy(hs_hbm.at[pl.ds(row_idx, 1)], row_v, sem).wait()

        accs = [jnp.zeros((_LANES,), jnp.float32) for _ in range(8)]
        for i in range(_NCHUNK):
            v = row_v[0, pl.ds(i * _LANES, _LANES)]
            accs[i % 8] = accs[i % 8] + v * v
        while len(accs) > 1:
            accs = [a + b for a, b in zip(accs[::2], accs[1::2])]
        ss = jnp.full((_LANES,), jnp.sum(accs[0]), jnp.float32)

        seed = jnp.int32(0x5F3759DF) - (plsc.bitcast(ss, jnp.int32) >> 1)
        y = plsc.bitcast(seed, jnp.float32)
        for _unused in range(3):
            y = y * (1.5 - 0.5 * ss * y * y)
        norm = ss * y
        scale_v = jnp.where(norm > 1e-12, y, jnp.float32(1e12))

        for i in range(_NCHUNK):
            sl = pl.ds(i * _LANES, _LANES)
            row_v[0, sl] = row_v[0, sl] * scale_v
        pltpu.sync_copy(row_v, out_hbm.at[pl.ds(r, 1)])


def kernel(hidden_states, extend_seq_lens):
    mesh = plsc.VectorSubcoreMesh(
        core_axis_name="c", subcore_axis_name="s", num_cores=2, num_subcores=16
    )
    f = pl.kernel(
        _pooler_body,
        out_type=jax.ShapeDtypeStruct((_BATCH, _D_MODEL), jnp.float32),
        mesh=mesh,
        scratch_types=[
            pltpu.VMEM((_LANES,), jnp.int32),
            pltpu.VMEM((1, _D_MODEL), jnp.float32),
            pltpu.SemaphoreType.DMA,
        ],
        compiler_params=pltpu.CompilerParams(needs_layout_passes=False),
    )
    return f(hidden_states, extend_seq_lens)


# speedup vs baseline: 1.0270x; 1.0050x over previous
import jax
import jax.numpy as jnp
from jax import lax
from jax.experimental import pallas as pl
from jax.experimental.pallas import tpu as pltpu
from jax.experimental.pallas import tpu_sc as plsc

_TOTAL_TOKENS = 32768
_BATCH = 16
_D_MODEL = 4096
_LANES = 16
_NCHUNK = _D_MODEL // _LANES
_ROWS_PER_CORE = _BATCH // 2


def _pooler_body(hs_hbm, lens_hbm, out_hbm, lens_v, row_v, sem):
    c = lax.axis_index("c")
    s = lax.axis_index("s")

    @pl.when(s < _ROWS_PER_CORE)
    def _():
        r = c * _ROWS_PER_CORE + s
        pltpu.sync_copy(lens_hbm, lens_v)
        last_idx = plsc.cumsum(lens_v[...]) - 1
        lane = lax.iota(jnp.int32, _LANES)
        row_idx = jnp.sum(jnp.where(lane == r, last_idx, 0))

        pltpu.async_copy(hs_hbm.at[pl.ds(row_idx, 1)], row_v, sem).wait()

        accs = [jnp.zeros((_LANES,), jnp.float32) for _ in range(8)]
        for i in range(_NCHUNK):
            v = row_v[0, pl.ds(i * _LANES, _LANES)]
            accs[i % 8] = accs[i % 8] + v * v
        while len(accs) > 1:
            accs = [a + b for a, b in zip(accs[::2], accs[1::2])]
        ss = jnp.full((_LANES,), jnp.sum(accs[0]), jnp.float32)

        seed = jnp.int32(0x5F3759DF) - (plsc.bitcast(ss, jnp.int32) >> 1)
        y = plsc.bitcast(seed, jnp.float32)
        for _unused in range(3):
            y = y * (1.5 - 0.5 * ss * y * y)
        norm = ss * y
        scale_v = jnp.where(norm > 1e-12, y, jnp.float32(1e12))

        for i in range(_NCHUNK):
            sl = pl.ds(i * _LANES, _LANES)
            row_v[0, sl] = row_v[0, sl] * scale_v
        pltpu.sync_copy(row_v, out_hbm.at[pl.ds(r, 1)])


def kernel(hidden_states, extend_seq_lens):
    mesh = plsc.VectorSubcoreMesh(
        core_axis_name="c", subcore_axis_name="s", num_cores=2, num_subcores=16
    )
    f = pl.kernel(
        _pooler_body,
        out_type=jax.ShapeDtypeStruct((_BATCH, _D_MODEL), jnp.float32),
        mesh=mesh,
        scratch_types=[
            pltpu.VMEM((_LANES,), jnp.int32),
            pltpu.VMEM((1, _D_MODEL), jnp.float32),
            pltpu.SemaphoreType.DMA,
        ],
        compiler_params=pltpu.CompilerParams(
            needs_layout_passes=False,
            skip_device_barrier=True,
        ),
    )
    return f(hidden_states, extend_seq_lens)


# speedup vs baseline: 8.8429x; 8.6101x over previous
import jax
import jax.numpy as jnp
from jax.experimental import pallas as pl
from jax.experimental.pallas import tpu as pltpu

_TOTAL_TOKENS = 32768
_BATCH = 16
_D_MODEL = 4096


def _pooler_body(lens_ref, hs_ref, out_ref, sems):
    copies = []
    running = lens_ref[0]
    for i in range(_BATCH):
        c = pltpu.make_async_copy(
            hs_ref.at[pl.ds(running - 1, 1)], out_ref.at[pl.ds(i, 1)], sems.at[i]
        )
        c.start()
        copies.append(c)
        if i + 1 < _BATCH:
            running = running + lens_ref[i + 1]
    for c in copies:
        c.wait()

    x = out_ref[...]
    ss = jnp.sum(x * x, axis=1, keepdims=True)
    norm = jnp.sqrt(ss)
    scale = jnp.where(norm > 1e-12, jax.lax.rsqrt(ss), 1e12)
    out_ref[...] = x * scale


def kernel(hidden_states, extend_seq_lens):
    return pl.pallas_call(
        _pooler_body,
        out_shape=jax.ShapeDtypeStruct((_BATCH, _D_MODEL), jnp.float32),
        in_specs=[
            pl.BlockSpec(memory_space=pltpu.SMEM),
            pl.BlockSpec(memory_space=pltpu.HBM),
        ],
        out_specs=pl.BlockSpec(memory_space=pltpu.VMEM),
        scratch_shapes=[pltpu.SemaphoreType.DMA((_BATCH,))],
    )(extend_seq_lens, hidden_states)
